# baseline (speedup 1.0000x reference)
import functools

import jax
import jax.numpy as jnp
from jax import lax
from jax.experimental import pallas as pl
from jax.experimental.pallas import tpu as pltpu
from jax.experimental.pallas import tpu_sc as plsc

BATCH = 16384
C_IN = 128
F_OUT = 512
LANES = 16

NUM_CORES = 2
NUM_SUBCORES = 16
NUM_WORKERS = NUM_CORES * NUM_SUBCORES
ROWS_PER_WORKER = BATCH // NUM_WORKERS
CHUNK_ROWS = 64
NUM_CHUNKS = ROWS_PER_WORKER // CHUNK_ROWS


def _sc_scatter_body(x_hbm, sel_hbm, out_hbm, sel_v,
                     x_v0, x_v1, out_v0, out_v1,
                     sem_x0, sem_x1, sem_o0, sem_o1):
    wid = lax.axis_index("s") * NUM_CORES + lax.axis_index("c")
    base = wid * ROWS_PER_WORKER
    x_bufs, out_bufs = (x_v0, x_v1), (out_v0, out_v1)
    sem_x, sem_o = (sem_x0, sem_x1), (sem_o0, sem_o1)

    def x_rows(k):
        return x_hbm.at[pl.ds(base + k * CHUNK_ROWS, CHUNK_ROWS)]

    def out_rows(k):
        return out_hbm.at[pl.ds(base + k * CHUNK_ROWS, CHUNK_ROWS)]

    pltpu.async_copy(x_rows(0), x_bufs[0], sem_x[0])
    pltpu.async_copy(x_rows(1), x_bufs[1], sem_x[1])

    pltpu.sync_copy(sel_hbm, sel_v)
    sel_regs = [sel_v[pl.ds(g * LANES, LANES)] for g in range(C_IN // LANES)]

    zeros = jnp.zeros((LANES,), jnp.float32)

    def zero_tile(tile):
        def zero_row(r, carry):
            for j in range(F_OUT // LANES):
                tile[r, pl.ds(j * LANES, LANES)] = zeros
            return carry

        lax.fori_loop(0, CHUNK_ROWS, zero_row, 0)

    def scatter_chunk(b, k):
        def scatter_row(r, c2):
            ridx = jnp.full((LANES,), r, jnp.int32)
            for g in range(C_IN // LANES):
                v = x_bufs[b][r, pl.ds(g * LANES, LANES)]
                plsc.store_scatter(out_bufs[b], [ridx, sel_regs[g]], v)
            return c2

        lax.fori_loop(0, CHUNK_ROWS, scatter_row, 0)

    zero_tile(out_v0)
    pltpu.make_async_copy(x_rows(0), x_bufs[0], sem_x[0]).wait()
    scatter_chunk(0, 0)
    pltpu.async_copy(out_bufs[0], out_rows(0), sem_o[0])
    pltpu.async_copy(x_rows(2), x_bufs[0], sem_x[0])
    zero_tile(out_v1)
    pltpu.make_async_copy(x_rows(1), x_bufs[1], sem_x[1]).wait()
    scatter_chunk(1, 1)
    pltpu.async_copy(out_bufs[1], out_rows(1), sem_o[1])
    pltpu.async_copy(x_rows(3), x_bufs[1], sem_x[1])

    def pair(p, carry):
        for j in range(2):
            k = 2 * p + j
            pltpu.make_async_copy(x_rows(k), x_bufs[j], sem_x[j]).wait()
            pltpu.make_async_copy(
                out_bufs[j], out_rows(k - 2), sem_o[j]).wait()
            scatter_chunk(j, k)
            pltpu.async_copy(out_bufs[j], out_rows(k), sem_o[j])

            @pl.when(k + 2 < NUM_CHUNKS)
            def _():
                pltpu.async_copy(x_rows(k + 2), x_bufs[j], sem_x[j])
        return carry

    lax.fori_loop(1, NUM_CHUNKS // 2, pair, 0)

    pltpu.make_async_copy(
        out_bufs[0], out_rows(NUM_CHUNKS - 2), sem_o[0]).wait()
    pltpu.make_async_copy(
        out_bufs[1], out_rows(NUM_CHUNKS - 1), sem_o[1]).wait()


@jax.jit
def kernel(x, sel):
    k = functools.partial(
        pl.kernel,
        mesh=plsc.VectorSubcoreMesh(core_axis_name="c", subcore_axis_name="s"),
        out_type=jax.ShapeDtypeStruct((BATCH, F_OUT), jnp.float32),
        compiler_params=pltpu.CompilerParams(needs_layout_passes=False),
        scratch_types=[
            pltpu.VMEM((C_IN,), jnp.int32),
            pltpu.VMEM((CHUNK_ROWS, C_IN), jnp.float32),
            pltpu.VMEM((CHUNK_ROWS, C_IN), jnp.float32),
            pltpu.VMEM((CHUNK_ROWS, F_OUT), jnp.float32),
            pltpu.VMEM((CHUNK_ROWS, F_OUT), jnp.float32),
            pltpu.SemaphoreType.DMA,
            pltpu.SemaphoreType.DMA,
            pltpu.SemaphoreType.DMA,
            pltpu.SemaphoreType.DMA,
        ],
    )(_sc_scatter_body)
    return k(x, sel)


# speedup vs baseline: 1.0122x; 1.0122x over previous
import functools

import jax
import jax.numpy as jnp
from jax import lax
from jax.experimental import pallas as pl
from jax.experimental.pallas import tpu as pltpu
from jax.experimental.pallas import tpu_sc as plsc

BATCH = 16384
C_IN = 128
F_OUT = 512
LANES = 16

NUM_CORES = 2
NUM_SUBCORES = 16
NUM_WORKERS = NUM_CORES * NUM_SUBCORES
ROWS_PER_WORKER = BATCH // NUM_WORKERS
CHUNK_ROWS = 64
NUM_CHUNKS = ROWS_PER_WORKER // CHUNK_ROWS


def _sc_scatter_body(x_hbm, sel_hbm, out_hbm, sel_v,
                     x_v0, x_v1, out_v0, out_v1,
                     sem_x0, sem_x1, sem_o0, sem_o1):
    wid = lax.axis_index("s") * NUM_CORES + lax.axis_index("c")
    base = wid * ROWS_PER_WORKER
    x_bufs, out_bufs = (x_v0, x_v1), (out_v0, out_v1)
    sem_x, sem_o = (sem_x0, sem_x1), (sem_o0, sem_o1)

    def x_rows(k):
        return x_hbm.at[pl.ds(base + k * CHUNK_ROWS, CHUNK_ROWS)]

    def out_rows(k):
        return out_hbm.at[pl.ds(base + k * CHUNK_ROWS, CHUNK_ROWS)]

    pltpu.async_copy(x_rows(0), x_bufs[0], sem_x[0])
    pltpu.async_copy(x_rows(1), x_bufs[1], sem_x[1])

    pltpu.sync_copy(sel_hbm, sel_v)
    sel_regs = [sel_v[pl.ds(g * LANES, LANES)] for g in range(C_IN // LANES)]

    zeros = jnp.zeros((LANES,), jnp.float32)

    def zero_row(r, carry):
        for j in range(F_OUT // LANES):
            out_v0[r, pl.ds(j * LANES, LANES)] = zeros
            out_v1[r, pl.ds(j * LANES, LANES)] = zeros
        return carry

    lax.fori_loop(0, CHUNK_ROWS, zero_row, 0)

    def scatter_rows(b, lo, hi):
        def scatter_row(r, c2):
            ridx = jnp.full((LANES,), r, jnp.int32)
            for g in range(C_IN // LANES):
                v = x_bufs[b][r, pl.ds(g * LANES, LANES)]
                plsc.store_scatter(out_bufs[b], [ridx, sel_regs[g]], v)
            return c2

        lax.fori_loop(lo, hi, scatter_row, 0)

    def pair(p, carry):
        for j in range(2):
            k = 2 * p + j
            pltpu.make_async_copy(x_rows(k), x_bufs[j], sem_x[j]).wait()

            @pl.when(k >= 2)
            def _():
                pltpu.make_async_copy(
                    out_bufs[j], out_rows(k - 2), sem_o[j]).wait()

            scatter_chunk = scatter_rows(j, 0, CHUNK_ROWS)
            pltpu.async_copy(out_bufs[j], out_rows(k), sem_o[j])

            @pl.when(k + 2 < NUM_CHUNKS)
            def _():
                pltpu.async_copy(x_rows(k + 2), x_bufs[j], sem_x[j])
        return carry

    lax.fori_loop(0, NUM_CHUNKS // 2 - 1, pair, 0)

    k6, k7 = NUM_CHUNKS - 2, NUM_CHUNKS - 1
    half = CHUNK_ROWS // 2
    pltpu.make_async_copy(x_rows(k6), x_bufs[0], sem_x[0]).wait()
    pltpu.make_async_copy(out_bufs[0], out_rows(k6 - 2), sem_o[0]).wait()
    scatter_rows(0, 0, CHUNK_ROWS)
    pltpu.async_copy(out_bufs[0], out_rows(k6), sem_o[0])

    pltpu.make_async_copy(x_rows(k7), x_bufs[1], sem_x[1]).wait()
    pltpu.make_async_copy(out_bufs[1], out_rows(k7 - 2), sem_o[1]).wait()
    scatter_rows(1, 0, half)
    pltpu.async_copy(
        out_bufs[1].at[pl.ds(0, half)],
        out_hbm.at[pl.ds(base + k7 * CHUNK_ROWS, half)], sem_o[1])
    scatter_rows(1, half, CHUNK_ROWS)
    pltpu.async_copy(
        out_bufs[1].at[pl.ds(half, half)],
        out_hbm.at[pl.ds(base + k7 * CHUNK_ROWS + half, half)], sem_o[1])

    pltpu.make_async_copy(out_bufs[0], out_rows(k6), sem_o[0]).wait()
    pltpu.make_async_copy(
        out_bufs[1].at[pl.ds(0, half)],
        out_hbm.at[pl.ds(base + k7 * CHUNK_ROWS, half)], sem_o[1]).wait()
    pltpu.make_async_copy(
        out_bufs[1].at[pl.ds(half, half)],
        out_hbm.at[pl.ds(base + k7 * CHUNK_ROWS + half, half)],
        sem_o[1]).wait()


@jax.jit
def kernel(x, sel):
    k = functools.partial(
        pl.kernel,
        mesh=plsc.VectorSubcoreMesh(core_axis_name="c", subcore_axis_name="s"),
        out_type=jax.ShapeDtypeStruct((BATCH, F_OUT), jnp.float32),
        compiler_params=pltpu.CompilerParams(needs_layout_passes=False),
        scratch_types=[
            pltpu.VMEM((C_IN,), jnp.int32),
            pltpu.VMEM((CHUNK_ROWS, C_IN), jnp.float32),
            pltpu.VMEM((CHUNK_ROWS, C_IN), jnp.float32),
            pltpu.VMEM((CHUNK_ROWS, F_OUT), jnp.float32),
            pltpu.VMEM((CHUNK_ROWS, F_OUT), jnp.float32),
            pltpu.SemaphoreType.DMA,
            pltpu.SemaphoreType.DMA,
            pltpu.SemaphoreType.DMA,
            pltpu.SemaphoreType.DMA,
        ],
    )(_sc_scatter_body)
    return k(x, sel)
